# baseline (speedup 1.0000x reference)
import jax
import jax.numpy as jnp
from jax import lax
from jax.experimental import pallas as pl
from jax.experimental.pallas import tpu as pltpu

N_Q = 1024
N_L = 16
D = 384
H = 64
N_ROLES = 1024
QB = 32
CHUNK = 128
NCH = N_ROLES // CHUNK
R = QB * N_L
GRID = N_Q // QB


def _body(qtrT_ref, llmsT_ref, remb_ref, wrole_ref, brole_ref,
          wqtrT_ref, wlT_ref, bqT_ref, u_ref, act_ref,
          sel_ref, lp_out_ref,
          rt_ref, lpT_ref, g2T_ref, gqT_ref, tri_ref):
    f32 = jnp.float32

    @pl.when(pl.program_id(0) == 0)
    def _init():
        row = lax.broadcasted_iota(jnp.int32, (CHUNK, CHUNK), 0)
        col = lax.broadcasted_iota(jnp.int32, (CHUNK, CHUNK), 1)
        tri_ref[...] = (col <= row).astype(f32)
        kk = lax.broadcasted_iota(jnp.int32, (QB + N_L, R), 0)
        rr = lax.broadcasted_iota(jnp.int32, (QB + N_L, R), 1)
        g2T_ref[...] = jnp.where(kk < QB, (kk == rr // N_L).astype(f32),
                                 (kk - QB == rr % N_L).astype(f32))
        rc = lax.broadcasted_iota(jnp.int32, (R, QB), 0)
        qq = lax.broadcasted_iota(jnp.int32, (R, QB), 1)
        gqT_ref[...] = (qq == rc // N_L).astype(f32)
        rt_un = (jnp.dot(remb_ref[...], wrole_ref[...],
                         preferred_element_type=f32) + brole_ref[...])
        nn = jnp.dot(rt_un * rt_un, jnp.ones((H, 1), f32),
                     preferred_element_type=f32)
        rt_ref[...] = rt_un * (1.0 / jnp.maximum(jnp.sqrt(nn), 1e-12))
        lpT_ref[...] = jnp.dot(wlT_ref[...], llmsT_ref[...],
                               preferred_element_type=f32)

    sT = (jnp.dot(wqtrT_ref[...], qtrT_ref[0],
                  preferred_element_type=f32) + bqT_ref[...])

    slpT = jnp.concatenate([sT, lpT_ref[...]], axis=1)
    g2T = g2T_ref[...]

    e2T = jnp.dot(slpT, g2T, preferred_element_type=f32)
    n2 = jnp.dot(jnp.ones((1, H), f32), e2T * e2T,
                 preferred_element_type=f32)
    inv = 1.0 / jnp.maximum(jnp.sqrt(n2), 1e-12)
    enT = e2T * inv

    eT = jnp.exp(jnp.dot(rt_ref[...], enT, preferred_element_type=f32))

    tri = tri_ref[...]
    fs = [jnp.dot(tri, eT[c * CHUNK:(c + 1) * CHUNK, :],
                  preferred_element_type=f32) for c in range(NCH)]
    carries = [jnp.zeros((1, R), f32)]
    for c in range(NCH):
        carries.append(carries[c] + fs[c][CHUNK - 1:CHUNK, :])
    z = carries[NCH]
    thr = u_ref[...] * z

    big = jnp.float32(3.0e38)
    cntv = jnp.zeros((CHUNK, R), f32)
    maxp = jnp.zeros((CHUNK, R), f32)
    minp = jnp.full((CHUNK, R), big, f32)
    for c in range(NCH):
        v = fs[c] + carries[c]
        cmp = v <= thr
        cntv = cntv + cmp.astype(f32)
        maxp = jnp.maximum(maxp, jnp.where(cmp, v, 0.0))
        minp = jnp.minimum(minp, jnp.where(cmp, big, v))
    cnt = jnp.dot(jnp.ones((1, CHUNK), f32), cntv,
                  preferred_element_type=f32).astype(jnp.int32)
    sel = jnp.where(cnt >= N_ROLES, 0, cnt)

    mx = jnp.max(maxp, axis=0, keepdims=True)
    mn = jnp.min(minp, axis=0, keepdims=True)
    sel_e = jnp.where(cnt >= N_ROLES, eT[0:1, :], mn - mx)

    logterm = (jnp.log(sel_e) - jnp.log(z)) * act_ref[...]

    sel_ref[...] = sel[None]
    lp_out_ref[...] = jnp.dot(logterm, gqT_ref[...],
                              preferred_element_type=f32)[None]


@jax.jit
def _run(qtrT, llmsT, role_emb, W_role, b_role2, wqtrT, wlT, bqT,
         uT, actT):
    rep = lambda shape: pl.BlockSpec(shape, lambda i: (0,) * len(shape))
    out = pl.pallas_call(
        _body,
        grid=(GRID,),
        in_specs=[
            pl.BlockSpec((1, 3 * D, QB), lambda i: (i, 0, 0)),
            rep((D, N_L)), rep((N_ROLES, D)), rep((D, H)), rep((1, H)),
            rep((H, 3 * D)), rep((H, D)), rep((H, 1)),
            pl.BlockSpec((1, R), lambda i: (0, i)),
            pl.BlockSpec((1, R), lambda i: (0, i)),
        ],
        out_specs=[
            pl.BlockSpec((1, 1, R), lambda i: (i, 0, 0)),
            pl.BlockSpec((1, 1, QB), lambda i: (i, 0, 0)),
        ],
        out_shape=[
            jax.ShapeDtypeStruct((GRID, 1, R), jnp.int32),
            jax.ShapeDtypeStruct((GRID, 1, QB), jnp.float32),
        ],
        scratch_shapes=[
            pltpu.VMEM((N_ROLES, H), jnp.float32),
            pltpu.VMEM((H, N_L), jnp.float32),
            pltpu.VMEM((QB + N_L, R), jnp.float32),
            pltpu.VMEM((R, QB), jnp.float32),
            pltpu.VMEM((CHUNK, CHUNK), jnp.float32),
        ],
    )(qtrT, llmsT, role_emb, W_role, b_role2, wqtrT, wlT, bqT, uT, actT)
    return out[0].reshape(N_Q, N_L), out[1].reshape(N_Q, 1)


def kernel(queries, tasks, llms_embedding, llms_num, reasonings, role_emb,
           W_qtlr, b_qtlr, W_role, b_role, rand_u):
    qtrT = jnp.concatenate([queries, tasks, reasonings], axis=1).T
    qtrT = qtrT.reshape(3 * D, GRID, QB).transpose(1, 0, 2)
    wqtrT = jnp.concatenate([W_qtlr[0:D], W_qtlr[D:2 * D],
                             W_qtlr[3 * D:4 * D]], axis=0).T
    wlT = W_qtlr[2 * D:3 * D].T
    llmsT = llms_embedding.T
    b_role2 = b_role.reshape(1, H)
    bqT = b_qtlr.reshape(H, 1)
    uT = rand_u.reshape(1, N_Q * N_L)
    actT = (llms_num > 0).astype(jnp.float32).reshape(1, N_Q * N_L)
    return _run(qtrT, llmsT, role_emb, W_role, b_role2, wqtrT, wlT, bqT,
                uT, actT)


# speedup vs baseline: 1.0854x; 1.0854x over previous
import jax
import jax.numpy as jnp
from jax import lax
from jax.experimental import pallas as pl
from jax.experimental.pallas import tpu as pltpu

N_Q = 1024
N_L = 16
D = 384
H = 64
N_ROLES = 1024
QB = 32
CHUNK = 128
NCH = N_ROLES // CHUNK
R = QB * N_L
GRID = N_Q // QB


def _body(qtrT_ref, llmsT_ref, remb_ref, wrole_ref, brole_ref,
          wqtrT_ref, wlT_ref, bqT_ref, u_ref, act_ref,
          sel_ref, lp_out_ref,
          rt_ref, lpT_ref, g2T_ref, gqT_ref, tri_ref):
    f32 = jnp.float32

    @pl.when(pl.program_id(0) == 0)
    def _init():
        row = lax.broadcasted_iota(jnp.int32, (CHUNK, CHUNK), 0)
        col = lax.broadcasted_iota(jnp.int32, (CHUNK, CHUNK), 1)
        tri_ref[...] = (col <= row).astype(f32)
        kk = lax.broadcasted_iota(jnp.int32, (QB + N_L, R), 0)
        rr = lax.broadcasted_iota(jnp.int32, (QB + N_L, R), 1)
        g2T_ref[...] = jnp.where(kk < QB, (kk == rr // N_L).astype(f32),
                                 (kk - QB == rr % N_L).astype(f32))
        rc = lax.broadcasted_iota(jnp.int32, (R, QB), 0)
        qq = lax.broadcasted_iota(jnp.int32, (R, QB), 1)
        gqT_ref[...] = (qq == rc // N_L).astype(f32)
        rt_un = (jnp.dot(remb_ref[...], wrole_ref[...],
                         preferred_element_type=f32) + brole_ref[...])
        nn = jnp.dot(rt_un * rt_un, jnp.ones((H, 1), f32),
                     preferred_element_type=f32)
        rt_ref[...] = rt_un * (1.0 / jnp.maximum(jnp.sqrt(nn), 1e-12))
        lpT_ref[...] = jnp.dot(wlT_ref[...], llmsT_ref[...],
                               preferred_element_type=f32)

    sT = (jnp.dot(wqtrT_ref[...], qtrT_ref[0],
                  preferred_element_type=f32) + bqT_ref[...])

    slpT = jnp.concatenate([sT, lpT_ref[...]], axis=1)
    g2T = g2T_ref[...]

    e2T = jnp.dot(slpT, g2T, preferred_element_type=f32)
    n2 = jnp.dot(jnp.ones((1, H), f32), e2T * e2T,
                 preferred_element_type=f32)
    inv = 1.0 / jnp.maximum(jnp.sqrt(n2), 1e-12)
    enT = e2T * inv

    eT = jnp.exp(jnp.dot(rt_ref[...], enT, preferred_element_type=f32))

    tri = tri_ref[...]
    fs = [jnp.dot(tri, eT[c * CHUNK:(c + 1) * CHUNK, :],
                  preferred_element_type=f32) for c in range(NCH)]
    carries = [jnp.zeros((1, R), f32)]
    for c in range(NCH):
        carries.append(carries[c] + fs[c][CHUNK - 1:CHUNK, :])
    z = carries[NCH]
    thr = u_ref[...] * z

    cntv = jnp.zeros((CHUNK, R), f32)
    for c in range(NCH):
        cntv = cntv + (fs[c] <= thr - carries[c]).astype(f32)
    cnt = jnp.dot(jnp.ones((1, CHUNK), f32), cntv,
                  preferred_element_type=f32).astype(jnp.int32)
    sel = jnp.where(cnt >= N_ROLES, 0, cnt)

    iota = lax.broadcasted_iota(jnp.int32, (N_ROLES, R), 0)
    masked = jnp.where(iota == sel, eT, 0.0)
    sel_e = jnp.dot(jnp.ones((1, N_ROLES), f32), masked,
                    preferred_element_type=f32)

    logterm = (jnp.log(sel_e) - jnp.log(z)) * act_ref[...]

    sel_ref[...] = sel[None]
    lp_out_ref[...] = jnp.dot(logterm, gqT_ref[...],
                              preferred_element_type=f32)[None]


@jax.jit
def _run(qtrT, llmsT, role_emb, W_role, b_role2, wqtrT, wlT, bqT,
         uT, actT):
    rep = lambda shape: pl.BlockSpec(shape, lambda i: (0,) * len(shape))
    out = pl.pallas_call(
        _body,
        grid=(GRID,),
        in_specs=[
            pl.BlockSpec((1, 3 * D, QB), lambda i: (i, 0, 0)),
            rep((D, N_L)), rep((N_ROLES, D)), rep((D, H)), rep((1, H)),
            rep((H, 3 * D)), rep((H, D)), rep((H, 1)),
            pl.BlockSpec((1, R), lambda i: (0, i)),
            pl.BlockSpec((1, R), lambda i: (0, i)),
        ],
        out_specs=[
            pl.BlockSpec((1, 1, R), lambda i: (i, 0, 0)),
            pl.BlockSpec((1, 1, QB), lambda i: (i, 0, 0)),
        ],
        out_shape=[
            jax.ShapeDtypeStruct((GRID, 1, R), jnp.int32),
            jax.ShapeDtypeStruct((GRID, 1, QB), jnp.float32),
        ],
        scratch_shapes=[
            pltpu.VMEM((N_ROLES, H), jnp.float32),
            pltpu.VMEM((H, N_L), jnp.float32),
            pltpu.VMEM((QB + N_L, R), jnp.float32),
            pltpu.VMEM((R, QB), jnp.float32),
            pltpu.VMEM((CHUNK, CHUNK), jnp.float32),
        ],
    )(qtrT, llmsT, role_emb, W_role, b_role2, wqtrT, wlT, bqT, uT, actT)
    return out[0].reshape(N_Q, N_L), out[1].reshape(N_Q, 1)


def kernel(queries, tasks, llms_embedding, llms_num, reasonings, role_emb,
           W_qtlr, b_qtlr, W_role, b_role, rand_u):
    qtrT = jnp.concatenate([queries, tasks, reasonings], axis=1).T
    qtrT = qtrT.reshape(3 * D, GRID, QB).transpose(1, 0, 2)
    wqtrT = jnp.concatenate([W_qtlr[0:D], W_qtlr[D:2 * D],
                             W_qtlr[3 * D:4 * D]], axis=0).T
    wlT = W_qtlr[2 * D:3 * D].T
    llmsT = llms_embedding.T
    b_role2 = b_role.reshape(1, H)
    bqT = b_qtlr.reshape(H, 1)
    uT = rand_u.reshape(1, N_Q * N_L)
    actT = (llms_num > 0).astype(jnp.float32).reshape(1, N_Q * N_L)
    return _run(qtrT, llmsT, role_emb, W_role, b_role2, wqtrT, wlT, bqT,
                uT, actT)


# speedup vs baseline: 1.4291x; 1.3166x over previous
import jax
import jax.numpy as jnp
from jax import lax
from jax.experimental import pallas as pl
from jax.experimental.pallas import tpu as pltpu

N_Q = 1024
N_L = 16
D = 384
H = 64
N_ROLES = 1024
QB = 64
CHUNK = 128
NCH = N_ROLES // CHUNK
R = QB * N_L
GRID = N_Q // QB


def _body(qtrT_ref, llmsT_ref, remb_ref, wrole_ref, brole_ref,
          wqtrT_ref, wlT_ref, bqT_ref, u_ref, act_ref,
          sel_ref, lp_out_ref,
          rt_ref, lpT_ref, g2T_ref, gqT_ref, tri_ref):
    f32 = jnp.float32

    @pl.when(pl.program_id(0) == 0)
    def _init():
        row = lax.broadcasted_iota(jnp.int32, (CHUNK, CHUNK), 0)
        col = lax.broadcasted_iota(jnp.int32, (CHUNK, CHUNK), 1)
        tri_ref[...] = (col <= row).astype(f32)
        kk = lax.broadcasted_iota(jnp.int32, (QB + N_L, R), 0)
        rr = lax.broadcasted_iota(jnp.int32, (QB + N_L, R), 1)
        g2T_ref[...] = jnp.where(kk < QB, (kk == rr // N_L).astype(f32),
                                 (kk - QB == rr % N_L).astype(f32))
        rc = lax.broadcasted_iota(jnp.int32, (R, QB), 0)
        qq = lax.broadcasted_iota(jnp.int32, (R, QB), 1)
        gqT_ref[...] = (qq == rc // N_L).astype(f32)
        rt_un = (jnp.dot(remb_ref[...], wrole_ref[...],
                         preferred_element_type=f32) + brole_ref[...])
        nn = jnp.dot(rt_un * rt_un, jnp.ones((H, 1), f32),
                     preferred_element_type=f32)
        rt_ref[...] = rt_un * (1.0 / jnp.maximum(jnp.sqrt(nn), 1e-12))
        lpT_ref[...] = jnp.dot(wlT_ref[...], llmsT_ref[...],
                               preferred_element_type=f32)

    sT = (jnp.dot(wqtrT_ref[...], qtrT_ref[0],
                  preferred_element_type=f32) + bqT_ref[...])

    slpT = jnp.concatenate([sT, lpT_ref[...]], axis=1)
    g2T = g2T_ref[...]

    e2T = jnp.dot(slpT, g2T, preferred_element_type=f32)
    n2 = jnp.dot(jnp.ones((1, H), f32), e2T * e2T,
                 preferred_element_type=f32)
    inv = 1.0 / jnp.maximum(jnp.sqrt(n2), 1e-12)
    enT = e2T * inv

    eT = jnp.exp(jnp.dot(rt_ref[...], enT, preferred_element_type=f32))

    tri = tri_ref[...]
    fs = [jnp.dot(tri, eT[c * CHUNK:(c + 1) * CHUNK, :],
                  preferred_element_type=f32) for c in range(NCH)]
    carries = [jnp.zeros((1, R), f32)]
    for c in range(NCH):
        carries.append(carries[c] + fs[c][CHUNK - 1:CHUNK, :])
    z = carries[NCH]
    thr = u_ref[...] * z

    cntv = jnp.zeros((CHUNK, R), f32)
    for c in range(NCH):
        cntv = cntv + (fs[c] <= thr - carries[c]).astype(f32)
    cnt = jnp.dot(jnp.ones((1, CHUNK), f32), cntv,
                  preferred_element_type=f32).astype(jnp.int32)
    sel = jnp.where(cnt >= N_ROLES, 0, cnt)

    iota = lax.broadcasted_iota(jnp.int32, (N_ROLES, R), 0)
    masked = jnp.where(iota == sel, eT, 0.0)
    sel_e = jnp.dot(jnp.ones((1, N_ROLES), f32), masked,
                    preferred_element_type=f32)

    logterm = (jnp.log(sel_e) - jnp.log(z)) * act_ref[...]

    sel_ref[...] = sel[None]
    lp_out_ref[...] = jnp.dot(logterm, gqT_ref[...],
                              preferred_element_type=f32)[None]


@jax.jit
def _run(qtrT, llmsT, role_emb, W_role, b_role2, wqtrT, wlT, bqT,
         uT, actT):
    rep = lambda shape: pl.BlockSpec(shape, lambda i: (0,) * len(shape))
    out = pl.pallas_call(
        _body,
        grid=(GRID,),
        in_specs=[
            pl.BlockSpec((1, 3 * D, QB), lambda i: (i, 0, 0)),
            rep((D, N_L)), rep((N_ROLES, D)), rep((D, H)), rep((1, H)),
            rep((H, 3 * D)), rep((H, D)), rep((H, 1)),
            pl.BlockSpec((1, R), lambda i: (0, i)),
            pl.BlockSpec((1, R), lambda i: (0, i)),
        ],
        out_specs=[
            pl.BlockSpec((1, 1, R), lambda i: (i, 0, 0)),
            pl.BlockSpec((1, 1, QB), lambda i: (i, 0, 0)),
        ],
        out_shape=[
            jax.ShapeDtypeStruct((GRID, 1, R), jnp.int32),
            jax.ShapeDtypeStruct((GRID, 1, QB), jnp.float32),
        ],
        scratch_shapes=[
            pltpu.VMEM((N_ROLES, H), jnp.float32),
            pltpu.VMEM((H, N_L), jnp.float32),
            pltpu.VMEM((QB + N_L, R), jnp.float32),
            pltpu.VMEM((R, QB), jnp.float32),
            pltpu.VMEM((CHUNK, CHUNK), jnp.float32),
        ],
    )(qtrT, llmsT, role_emb, W_role, b_role2, wqtrT, wlT, bqT, uT, actT)
    return out[0].reshape(N_Q, N_L), out[1].reshape(N_Q, 1)


def kernel(queries, tasks, llms_embedding, llms_num, reasonings, role_emb,
           W_qtlr, b_qtlr, W_role, b_role, rand_u):
    qtrT = jnp.concatenate([queries, tasks, reasonings], axis=1).T
    qtrT = qtrT.reshape(3 * D, GRID, QB).transpose(1, 0, 2)
    wqtrT = jnp.concatenate([W_qtlr[0:D], W_qtlr[D:2 * D],
                             W_qtlr[3 * D:4 * D]], axis=0).T
    wlT = W_qtlr[2 * D:3 * D].T
    llmsT = llms_embedding.T
    b_role2 = b_role.reshape(1, H)
    bqT = b_qtlr.reshape(H, 1)
    uT = rand_u.reshape(1, N_Q * N_L)
    actT = (llms_num > 0).astype(jnp.float32).reshape(1, N_Q * N_L)
    return _run(qtrT, llmsT, role_emb, W_role, b_role2, wqtrT, wlT, bqT,
                uT, actT)


# speedup vs baseline: 1.7682x; 1.2373x over previous
import jax
import jax.numpy as jnp
from jax import lax
from jax.experimental import pallas as pl
from jax.experimental.pallas import tpu as pltpu

N_Q = 1024
N_L = 16
D = 384
H = 64
N_ROLES = 1024
QB = 128
CHUNK = 128
NCH = N_ROLES // CHUNK
R = QB * N_L
GRID = N_Q // QB


def _body(qtrT_ref, llmsT_ref, remb_ref, wrole_ref, brole_ref,
          wqtrT_ref, wlT_ref, bqT_ref, u_ref, act_ref,
          sel_ref, lp_out_ref,
          rt_ref, lpT_ref, g2T_ref, gqT_ref, tri_ref):
    f32 = jnp.float32

    @pl.when(pl.program_id(0) == 0)
    def _init():
        row = lax.broadcasted_iota(jnp.int32, (CHUNK, CHUNK), 0)
        col = lax.broadcasted_iota(jnp.int32, (CHUNK, CHUNK), 1)
        tri_ref[...] = (col <= row).astype(f32)
        kk = lax.broadcasted_iota(jnp.int32, (QB + N_L, R), 0)
        rr = lax.broadcasted_iota(jnp.int32, (QB + N_L, R), 1)
        g2T_ref[...] = jnp.where(kk < QB, (kk == rr // N_L).astype(f32),
                                 (kk - QB == rr % N_L).astype(f32))
        rc = lax.broadcasted_iota(jnp.int32, (R, QB), 0)
        qq = lax.broadcasted_iota(jnp.int32, (R, QB), 1)
        gqT_ref[...] = (qq == rc // N_L).astype(f32)
        rt_un = (jnp.dot(remb_ref[...], wrole_ref[...],
                         preferred_element_type=f32) + brole_ref[...])
        nn = jnp.dot(rt_un * rt_un, jnp.ones((H, 1), f32),
                     preferred_element_type=f32)
        rt_ref[...] = rt_un * (1.0 / jnp.maximum(jnp.sqrt(nn), 1e-12))
        lpT_ref[...] = jnp.dot(wlT_ref[...], llmsT_ref[...],
                               preferred_element_type=f32)

    sT = (jnp.dot(wqtrT_ref[...], qtrT_ref[0],
                  preferred_element_type=f32) + bqT_ref[...])

    slpT = jnp.concatenate([sT, lpT_ref[...]], axis=1)
    g2T = g2T_ref[...]

    e2T = jnp.dot(slpT, g2T, preferred_element_type=f32)
    n2 = jnp.dot(jnp.ones((1, H), f32), e2T * e2T,
                 preferred_element_type=f32)
    inv = 1.0 / jnp.maximum(jnp.sqrt(n2), 1e-12)
    enT = e2T * inv

    eT = jnp.exp(jnp.dot(rt_ref[...], enT, preferred_element_type=f32))

    tri = tri_ref[...]
    fs = [jnp.dot(tri, eT[c * CHUNK:(c + 1) * CHUNK, :],
                  preferred_element_type=f32) for c in range(NCH)]
    carries = [jnp.zeros((1, R), f32)]
    for c in range(NCH):
        carries.append(carries[c] + fs[c][CHUNK - 1:CHUNK, :])
    z = carries[NCH]
    thr = u_ref[...] * z

    cntv = jnp.zeros((CHUNK, R), f32)
    for c in range(NCH):
        cntv = cntv + (fs[c] <= thr - carries[c]).astype(f32)
    cnt = jnp.dot(jnp.ones((1, CHUNK), f32), cntv,
                  preferred_element_type=f32).astype(jnp.int32)
    sel = jnp.where(cnt >= N_ROLES, 0, cnt)

    iota = lax.broadcasted_iota(jnp.int32, (N_ROLES, R), 0)
    masked = jnp.where(iota == sel, eT, 0.0)
    sel_e = jnp.dot(jnp.ones((1, N_ROLES), f32), masked,
                    preferred_element_type=f32)

    logterm = (jnp.log(sel_e) - jnp.log(z)) * act_ref[...]

    sel_ref[...] = sel[None]
    lp_out_ref[...] = jnp.dot(logterm, gqT_ref[...],
                              preferred_element_type=f32)[None]


@jax.jit
def _run(qtrT, llmsT, role_emb, W_role, b_role2, wqtrT, wlT, bqT,
         uT, actT):
    rep = lambda shape: pl.BlockSpec(shape, lambda i: (0,) * len(shape))
    out = pl.pallas_call(
        _body,
        grid=(GRID,),
        in_specs=[
            pl.BlockSpec((1, 3 * D, QB), lambda i: (i, 0, 0)),
            rep((D, N_L)), rep((N_ROLES, D)), rep((D, H)), rep((1, H)),
            rep((H, 3 * D)), rep((H, D)), rep((H, 1)),
            pl.BlockSpec((1, R), lambda i: (0, i)),
            pl.BlockSpec((1, R), lambda i: (0, i)),
        ],
        out_specs=[
            pl.BlockSpec((1, 1, R), lambda i: (i, 0, 0)),
            pl.BlockSpec((1, 1, QB), lambda i: (i, 0, 0)),
        ],
        out_shape=[
            jax.ShapeDtypeStruct((GRID, 1, R), jnp.int32),
            jax.ShapeDtypeStruct((GRID, 1, QB), jnp.float32),
        ],
        scratch_shapes=[
            pltpu.VMEM((N_ROLES, H), jnp.float32),
            pltpu.VMEM((H, N_L), jnp.float32),
            pltpu.VMEM((QB + N_L, R), jnp.float32),
            pltpu.VMEM((R, QB), jnp.float32),
            pltpu.VMEM((CHUNK, CHUNK), jnp.float32),
        ],
    )(qtrT, llmsT, role_emb, W_role, b_role2, wqtrT, wlT, bqT, uT, actT)
    return out[0].reshape(N_Q, N_L), out[1].reshape(N_Q, 1)


def kernel(queries, tasks, llms_embedding, llms_num, reasonings, role_emb,
           W_qtlr, b_qtlr, W_role, b_role, rand_u):
    qtrT = jnp.concatenate([queries, tasks, reasonings], axis=1).T
    qtrT = qtrT.reshape(3 * D, GRID, QB).transpose(1, 0, 2)
    wqtrT = jnp.concatenate([W_qtlr[0:D], W_qtlr[D:2 * D],
                             W_qtlr[3 * D:4 * D]], axis=0).T
    wlT = W_qtlr[2 * D:3 * D].T
    llmsT = llms_embedding.T
    b_role2 = b_role.reshape(1, H)
    bqT = b_qtlr.reshape(H, 1)
    uT = rand_u.reshape(1, N_Q * N_L)
    actT = (llms_num > 0).astype(jnp.float32).reshape(1, N_Q * N_L)
    return _run(qtrT, llmsT, role_emb, W_role, b_role2, wqtrT, wlT, bqT,
                uT, actT)
